# baseline (speedup 1.0000x reference)
import functools

import jax
import jax.numpy as jnp
from jax import lax
from jax.experimental import pallas as pl
from jax.experimental.pallas import tpu as pltpu
from jax.experimental.pallas import tpu_sc as plsc

N = 10000
D = 128
NP = 10240
EDGES = 330000
NC = 2
NS = 16
NWORK = NC * NS
GRP = 48
SG = 8
NSG = 27
NSG_A = 30
NSG_B = 24
TSG = NSG_A + NSG_B
NSG3 = NSG // 3
NGRP = SG * NSG
W_PER = NGRP * GRP
EP = NWORK * W_PER
OUT_PT = 624
NT = 10112
_ZCHUNKS = [(r, min(GRP, OUT_PT - r)) for r in range(0, OUT_PT, GRP)]
_SCHUNKS = [(r, min(GRP, 632 - r)) for r in range(0, 632, GRP)]

_f32 = jnp.float32
_i32 = jnp.int32


_ROWBLK = 5120
_NBLK = NP // _ROWBLK


def _dense_body(x_ref, w_ref, asv_ref, adv_ref, h_ref, as_ref, ad_ref):
    h = jnp.dot(x_ref[...], w_ref[...], preferred_element_type=_f32)
    h_ref[...] = h
    as_ref[...] = jnp.sum(h * asv_ref[...], axis=1)
    ad_ref[...] = jnp.sum(h * adv_ref[...], axis=1)


def _dense(x, w, asv, adv):
    return pl.pallas_call(
        _dense_body,
        grid=(_NBLK,),
        in_specs=[
            pl.BlockSpec((_ROWBLK, D), lambda i: (i, 0)),
            pl.BlockSpec((D, D), lambda i: (0, 0)),
            pl.BlockSpec((1, D), lambda i: (0, 0)),
            pl.BlockSpec((1, D), lambda i: (0, 0)),
        ],
        out_specs=[
            pl.BlockSpec((_ROWBLK, D), lambda i: (i, 0)),
            pl.BlockSpec((_ROWBLK,), lambda i: (i,)),
            pl.BlockSpec((_ROWBLK,), lambda i: (i,)),
        ],
        out_shape=[
            jax.ShapeDtypeStruct((NP, D), _f32),
            jax.ShapeDtypeStruct((NP,), _f32),
            jax.ShapeDtypeStruct((NP,), _f32),
        ],
    )(x, w, asv, adv)


def _combine_dense_body(parts_ref, sparts_ref, bias_ref, scale_ref, shift_ref,
                        w_ref, asv_ref, adv_ref, h_ref, as_ref, ad_ref):
    p = parts_ref[0] + parts_ref[1]
    s = sparts_ref[0] + sparts_ref[1]
    inv = 1.0 / (s + 1e-16)
    t = jnp.maximum(p * inv[:, None] + bias_ref[...], 0.0)
    t = t * scale_ref[...] + shift_ref[...]
    h = jnp.dot(t, w_ref[...], preferred_element_type=_f32)
    h_ref[...] = h
    as_ref[...] = jnp.sum(h * asv_ref[...], axis=1)
    ad_ref[...] = jnp.sum(h * adv_ref[...], axis=1)


def _combine_dense(parts, sparts, bias, scale, shift, w, asv, adv):
    return pl.pallas_call(
        _combine_dense_body,
        grid=(_NBLK,),
        in_specs=[
            pl.BlockSpec((NC, _ROWBLK, D), lambda i: (0, i, 0)),
            pl.BlockSpec((NC, _ROWBLK), lambda i: (0, i)),
            pl.BlockSpec((1, D), lambda i: (0, 0)),
            pl.BlockSpec((1, D), lambda i: (0, 0)),
            pl.BlockSpec((1, D), lambda i: (0, 0)),
            pl.BlockSpec((D, D), lambda i: (0, 0)),
            pl.BlockSpec((1, D), lambda i: (0, 0)),
            pl.BlockSpec((1, D), lambda i: (0, 0)),
        ],
        out_specs=[
            pl.BlockSpec((_ROWBLK, D), lambda i: (i, 0)),
            pl.BlockSpec((_ROWBLK,), lambda i: (i,)),
            pl.BlockSpec((_ROWBLK,), lambda i: (i,)),
        ],
        out_shape=[
            jax.ShapeDtypeStruct((NP, D), _f32),
            jax.ShapeDtypeStruct((NP,), _f32),
            jax.ShapeDtypeStruct((NP,), _f32),
        ],
    )(parts, sparts, bias, scale, shift, w, asv, adv)


def _final_body(parts_ref, sparts_ref, bias_ref, out_ref):
    p = parts_ref[0] + parts_ref[1]
    s = sparts_ref[0] + sparts_ref[1]
    inv = 1.0 / (s + 1e-16)
    out_ref[...] = jnp.maximum(p * inv[:, None] + bias_ref[...], 0.0)


def _final(parts, sparts, bias):
    return pl.pallas_call(
        _final_body,
        grid=(_NBLK,),
        in_specs=[
            pl.BlockSpec((NC, _ROWBLK, D), lambda i: (0, i, 0)),
            pl.BlockSpec((NC, _ROWBLK), lambda i: (0, i)),
            pl.BlockSpec((1, D), lambda i: (0, 0)),
        ],
        out_specs=pl.BlockSpec((_ROWBLK, D), lambda i: (i, 0)),
        out_shape=jax.ShapeDtypeStruct((NP, D), _f32),
    )(parts, sparts, bias)


def _edge_body(h_hbm, asrc_hbm, adst_hbm, srcsg_hbm, dstsg_hbm,
               sparts_hbm, parts_hbm,
               srcg, dstg, asrc_v, adst_v, rows, exb,
               out_sh, s_sh, sem_g, sem_sc, sem_ix, sem_ss):
    cid = lax.axis_index("c")
    sid = lax.axis_index("s")
    wid = sid * NC + cid
    nsg3 = jnp.where(cid == 0, NSG_A // 3, NSG_B // 3)
    last_i = nsg3 - 1
    base_sg = sid * TSG + cid * NSG_A

    pltpu.async_copy(asrc_hbm.at[pl.ds(0, NT)], asrc_v, sem_g[0])
    pltpu.async_copy(adst_hbm.at[pl.ds(0, NT)], adst_v, sem_g[1])

    zeros16 = jnp.zeros((16,), _f32)
    lanes = lax.iota(_i32, 16)
    e16s = [j2 * 16 + lanes for j2 in range(GRP // 16)]

    def _zr(c, cc):
        c16 = jnp.full((16,), c, _i32)
        for j2 in range(GRP // 16):
            plsc.store_scatter(rows[0], [e16s[j2], c16], zeros16)
        return cc

    lax.fori_loop(0, D, _zr, 0)
    for j2 in range(GRP // 16):
        exb[0][pl.ds(j2 * 16, 16)] = zeros16

    base_row = sid * OUT_PT
    for off, ln in _ZCHUNKS:
        pltpu.async_copy(rows[0].at[pl.ds(0, ln)],
                         out_sh.at[pl.ds(base_row + off, ln)], sem_sc[0])

    @pl.when(sid == NS - 1)
    def _ztail():
        pltpu.sync_copy(rows[0].at[pl.ds(0, 16)],
                        out_sh.at[pl.ds(NS * OUT_PT, 16)])

    s_base = sid * (NT // NS)
    for soff, sln in _SCHUNKS:
        pltpu.async_copy(exb[0].at[pl.ds(0, sln)],
                         s_sh.at[pl.ds(s_base + soff, sln)], sem_sc[1])

    for off, ln in _ZCHUNKS:
        pltpu.make_async_copy(rows[0].at[pl.ds(0, ln)],
                              out_sh.at[pl.ds(base_row + off, ln)],
                              sem_sc[0]).wait()
    for soff, sln in _SCHUNKS:
        pltpu.make_async_copy(exb[0].at[pl.ds(0, sln)],
                              s_sh.at[pl.ds(s_base + soff, sln)],
                              sem_sc[1]).wait()
    pltpu.make_async_copy(asrc_hbm.at[pl.ds(0, NT)], asrc_v, sem_g[0]).wait()
    pltpu.make_async_copy(adst_hbm.at[pl.ds(0, NT)], adst_v, sem_g[1]).wait()

    plsc.subcore_barrier()

    pltpu.sync_copy(srcsg_hbm.at[base_sg], srcg[0])
    pltpu.sync_copy(dstsg_hbm.at[base_sg], dstg[0])
    pltpu.async_copy(h_hbm.at[srcg[0].at[pl.ds(0, GRP)]], rows[0], sem_g[0])
    pltpu.async_copy(h_hbm.at[srcg[0].at[pl.ds(GRP, GRP)]], rows[1], sem_g[1])
    pltpu.async_copy(h_hbm.at[srcg[0].at[pl.ds(2 * GRP, GRP)]], rows[2],
                     sem_g[2])


    def _wait_scatter(q):
        pltpu.make_async_copy(rows[q], out_sh.at[dstg[0].at[0]],
                              sem_sc[q]).wait()

    def _wait_sscatter(q):
        pltpu.make_async_copy(exb[q], s_sh.at[dstg[0].at[0]],
                              sem_ss[q]).wait()

    def _body(i, carry):
        for s3 in range(3):
            b3 = s3
            nb3 = (s3 + 1) % 3
            s_dyn = 3 * i + s3
            for k in range(SG):
                b = k % 4
                g_dyn = s_dyn * SG + k
                ebase = base_sg * (SG * GRP)
                if s3 == 0 and k < 4:
                    @pl.when(i > 0)
                    def _():
                        _wait_sscatter(b)
                else:
                    _wait_sscatter(b)
                for j2 in range(GRP // 16):
                    s16 = srcg[b3][pl.ds(k * GRP + j2 * 16, 16)]
                    d16 = dstg[b3][k, pl.ds(j2 * 16, 16)]
                    av = plsc.load_gather(asrc_v, [s16])
                    bv = plsc.load_gather(adst_v, [d16])
                    e = av + bv
                    e = jnp.where(e > 0.0, e, 0.2 * e)
                    ex = jnp.exp(e)
                    eid = ebase + g_dyn * GRP + j2 * 16 + lanes
                    ex = jnp.where(eid < EDGES, ex, 0.0)
                    exb[b][pl.ds(j2 * 16, 16)] = ex
                pltpu.async_copy(exb[b], s_sh.at[dstg[b3].at[k]], sem_ss[b],
                                 add=True)
                if k == 0:
                    def _issue_idx():
                        pltpu.async_copy(srcsg_hbm.at[base_sg + s_dyn + 1],
                                         srcg[nb3], sem_ix[nb3])
                        pltpu.async_copy(dstsg_hbm.at[base_sg + s_dyn + 1],
                                         dstg[nb3], sem_ix[nb3])
                    if s3 == 2:
                        pl.when(i < last_i)(_issue_idx)
                    else:
                        _issue_idx()
                pltpu.make_async_copy(h_hbm.at[srcg[0].at[pl.ds(0, GRP)]],
                                      rows[b], sem_g[b]).wait()
                @plsc.parallel_loop(0, GRP, step=1, unroll=2)
                def _sc(e):
                    m = plsc.load_gather(exb[b], [jnp.full((16,), e, _i32)])
                    for j in range(D // 16):
                        rows[b][e, pl.ds(j * 16, 16)] = (
                            rows[b][e, pl.ds(j * 16, 16)] * m)
                pltpu.async_copy(rows[b], out_sh.at[dstg[b3].at[k]],
                                 sem_sc[b], add=True)
                if s3 == 0 and k == 0:
                    @pl.when(i > 0)
                    def _():
                        _wait_scatter(3)
                else:
                    _wait_scatter((k + 3) % 4)
                def _issue_gather():
                    if k == SG - 3:
                        pltpu.make_async_copy(srcsg_hbm.at[base_sg],
                                              srcg[nb3], sem_ix[nb3]).wait()
                        pltpu.make_async_copy(dstsg_hbm.at[base_sg],
                                              dstg[nb3], sem_ix[nb3]).wait()
                        idx_ref = srcg[nb3].at[pl.ds(0, GRP)]
                    elif k > SG - 3:
                        idx_ref = srcg[nb3].at[pl.ds((k - (SG - 3)) * GRP,
                                                     GRP)]
                    else:
                        idx_ref = srcg[b3].at[pl.ds((k + 3) * GRP, GRP)]
                    pltpu.async_copy(h_hbm.at[idx_ref], rows[(k + 3) % 4],
                                     sem_g[(k + 3) % 4])
                if s3 == 2 and k >= SG - 3:
                    pl.when(i < last_i)(_issue_gather)
                else:
                    _issue_gather()
        return carry

    lax.fori_loop(0, nsg3, _body, 0)

    for q in range(4):
        _wait_sscatter(q)
    _wait_scatter(3)
    plsc.subcore_barrier()

    @pl.when(sid == 0)
    def _ws():
        pltpu.sync_copy(s_sh, sparts_hbm.at[cid].at[pl.ds(0, NT)])

    for off, ln in _ZCHUNKS:
        pltpu.async_copy(out_sh.at[pl.ds(base_row + off, ln)],
                         parts_hbm.at[cid].at[pl.ds(base_row + off, ln)],
                         sem_sc[0])
    for off, ln in _ZCHUNKS:
        pltpu.make_async_copy(out_sh.at[pl.ds(base_row + off, ln)],
                              parts_hbm.at[cid].at[pl.ds(base_row + off, ln)],
                              sem_sc[0]).wait()

    @pl.when(sid == NS - 1)
    def _wtail():
        pltpu.sync_copy(out_sh.at[pl.ds(NS * OUT_PT, 16)],
                        parts_hbm.at[cid].at[pl.ds(NS * OUT_PT, 16)])


_edge_kernel = pl.kernel(
    _edge_body,
    out_type=[
        jax.ShapeDtypeStruct((NC, NP), _f32),
        jax.ShapeDtypeStruct((NC, NP, D), _f32),
    ],
    mesh=plsc.VectorSubcoreMesh(core_axis_name="c", subcore_axis_name="s"),
    compiler_params=pltpu.CompilerParams(needs_layout_passes=False),
    scratch_types=[
        [pltpu.VMEM((SG * GRP,), _i32) for _ in range(3)],
        [pltpu.VMEM((SG, GRP), _i32) for _ in range(3)],
        pltpu.VMEM((NT,), _f32),
        pltpu.VMEM((NT,), _f32),
        [pltpu.VMEM((GRP, D), _f32) for _ in range(4)],
        [pltpu.VMEM((GRP,), _f32) for _ in range(4)],
        pltpu.VMEM_SHARED((N, D), _f32),
        pltpu.VMEM_SHARED((NT,), _f32),
        [pltpu.SemaphoreType.DMA for _ in range(4)],
        [pltpu.SemaphoreType.DMA for _ in range(4)],
        [pltpu.SemaphoreType.DMA for _ in range(3)],
        [pltpu.SemaphoreType.DMA for _ in range(4)],
    ],
)


def kernel(x, adj, W1, att_src1, att_dst1, bias1, bn_gamma, bn_beta, bn_mean,
           bn_var, W2, att_src2, att_dst2, bias2):
    loop = jnp.arange(N, dtype=_i32)
    src = jnp.concatenate([adj[0].astype(_i32), loop])
    dst = jnp.concatenate([adj[1].astype(_i32), loop])
    src = jnp.pad(src, (0, EP - EDGES))
    dst = jnp.pad(dst, (0, EP - EDGES))
    srcsg = src.reshape(NS * TSG, SG * GRP)
    dstsg = dst.reshape(NS * TSG, SG, GRP)

    xp = jnp.pad(x, ((0, NP - N), (0, 0)))
    asv1 = att_src1.reshape(1, D)
    adv1 = att_dst1.reshape(1, D)
    asv2 = att_src2.reshape(1, D)
    adv2 = att_dst2.reshape(1, D)
    bn_scale = (bn_gamma / jnp.sqrt(bn_var + 1e-5)).reshape(1, D)
    bn_shift = bn_beta.reshape(1, D) - bn_mean.reshape(1, D) * bn_scale

    h1, as1, ad1 = _dense(xp, W1, asv1, adv1)
    sparts1, parts1 = _edge_kernel(h1, as1, ad1, srcsg, dstsg)
    h2, as2, ad2 = _combine_dense(parts1, sparts1, bias1.reshape(1, D),
                                  bn_scale, bn_shift, W2, asv2, adv2)
    sparts2, parts2 = _edge_kernel(h2, as2, ad2, srcsg, dstsg)
    out = _final(parts2, sparts2, bias2.reshape(1, D))
    return out[:N]


# speedup vs baseline: 1.0368x; 1.0368x over previous
import functools

import jax
import jax.numpy as jnp
from jax import lax
from jax.experimental import pallas as pl
from jax.experimental.pallas import tpu as pltpu
from jax.experimental.pallas import tpu_sc as plsc

N = 10000
D = 128
NP = 10240
EDGES = 330000
NC = 2
NS = 16
NWORK = NC * NS
GRP = 48
SG = 8
NSG = 27
NSG_A = 33
NSG_B = 21
TSG = NSG_A + NSG_B
NSG3 = NSG // 3
NGRP = SG * NSG
W_PER = NGRP * GRP
EP = NWORK * W_PER
OUT_PT = 624
NT = 10112
_ZCHUNKS = [(r, min(GRP, OUT_PT - r)) for r in range(0, OUT_PT, GRP)]
_SCHUNKS = [(r, min(GRP, 632 - r)) for r in range(0, 632, GRP)]

_f32 = jnp.float32
_i32 = jnp.int32


_ROWBLK = 5120
_NBLK = NP // _ROWBLK


def _dense_body(x_ref, w_ref, asv_ref, adv_ref, h_ref, as_ref, ad_ref):
    h = jnp.dot(x_ref[...], w_ref[...], preferred_element_type=_f32)
    h_ref[...] = h
    as_ref[...] = jnp.sum(h * asv_ref[...], axis=1)
    ad_ref[...] = jnp.sum(h * adv_ref[...], axis=1)


def _dense(x, w, asv, adv):
    return pl.pallas_call(
        _dense_body,
        grid=(_NBLK,),
        in_specs=[
            pl.BlockSpec((_ROWBLK, D), lambda i: (i, 0)),
            pl.BlockSpec((D, D), lambda i: (0, 0)),
            pl.BlockSpec((1, D), lambda i: (0, 0)),
            pl.BlockSpec((1, D), lambda i: (0, 0)),
        ],
        out_specs=[
            pl.BlockSpec((_ROWBLK, D), lambda i: (i, 0)),
            pl.BlockSpec((_ROWBLK,), lambda i: (i,)),
            pl.BlockSpec((_ROWBLK,), lambda i: (i,)),
        ],
        out_shape=[
            jax.ShapeDtypeStruct((NP, D), _f32),
            jax.ShapeDtypeStruct((NP,), _f32),
            jax.ShapeDtypeStruct((NP,), _f32),
        ],
    )(x, w, asv, adv)


def _combine_dense_body(parts_ref, sparts_ref, bias_ref, scale_ref, shift_ref,
                        w_ref, asv_ref, adv_ref, h_ref, as_ref, ad_ref):
    p = parts_ref[0] + parts_ref[1]
    s = sparts_ref[0] + sparts_ref[1]
    inv = 1.0 / (s + 1e-16)
    t = jnp.maximum(p * inv[:, None] + bias_ref[...], 0.0)
    t = t * scale_ref[...] + shift_ref[...]
    h = jnp.dot(t, w_ref[...], preferred_element_type=_f32)
    h_ref[...] = h
    as_ref[...] = jnp.sum(h * asv_ref[...], axis=1)
    ad_ref[...] = jnp.sum(h * adv_ref[...], axis=1)


def _combine_dense(parts, sparts, bias, scale, shift, w, asv, adv):
    return pl.pallas_call(
        _combine_dense_body,
        grid=(_NBLK,),
        in_specs=[
            pl.BlockSpec((NC, _ROWBLK, D), lambda i: (0, i, 0)),
            pl.BlockSpec((NC, _ROWBLK), lambda i: (0, i)),
            pl.BlockSpec((1, D), lambda i: (0, 0)),
            pl.BlockSpec((1, D), lambda i: (0, 0)),
            pl.BlockSpec((1, D), lambda i: (0, 0)),
            pl.BlockSpec((D, D), lambda i: (0, 0)),
            pl.BlockSpec((1, D), lambda i: (0, 0)),
            pl.BlockSpec((1, D), lambda i: (0, 0)),
        ],
        out_specs=[
            pl.BlockSpec((_ROWBLK, D), lambda i: (i, 0)),
            pl.BlockSpec((_ROWBLK,), lambda i: (i,)),
            pl.BlockSpec((_ROWBLK,), lambda i: (i,)),
        ],
        out_shape=[
            jax.ShapeDtypeStruct((NP, D), _f32),
            jax.ShapeDtypeStruct((NP,), _f32),
            jax.ShapeDtypeStruct((NP,), _f32),
        ],
    )(parts, sparts, bias, scale, shift, w, asv, adv)


def _final_body(parts_ref, sparts_ref, bias_ref, out_ref):
    p = parts_ref[0] + parts_ref[1]
    s = sparts_ref[0] + sparts_ref[1]
    inv = 1.0 / (s + 1e-16)
    out_ref[...] = jnp.maximum(p * inv[:, None] + bias_ref[...], 0.0)


def _final(parts, sparts, bias):
    return pl.pallas_call(
        _final_body,
        grid=(_NBLK,),
        in_specs=[
            pl.BlockSpec((NC, _ROWBLK, D), lambda i: (0, i, 0)),
            pl.BlockSpec((NC, _ROWBLK), lambda i: (0, i)),
            pl.BlockSpec((1, D), lambda i: (0, 0)),
        ],
        out_specs=pl.BlockSpec((_ROWBLK, D), lambda i: (i, 0)),
        out_shape=jax.ShapeDtypeStruct((NP, D), _f32),
    )(parts, sparts, bias)


def _edge_body(h_hbm, asrc_hbm, adst_hbm, srcsg_hbm, dstsg_hbm,
               sparts_hbm, parts_hbm,
               srcg, dstg, asrc_v, adst_v, rows, exb,
               out_sh, s_sh, sem_g, sem_sc, sem_ix, sem_ss):
    cid = lax.axis_index("c")
    sid = lax.axis_index("s")
    wid = sid * NC + cid
    nsg3 = jnp.where(cid == 0, NSG_A // 3, NSG_B // 3)
    last_i = nsg3 - 1
    base_sg = sid * TSG + cid * NSG_A

    pltpu.async_copy(asrc_hbm.at[pl.ds(0, NT)], asrc_v, sem_g[0])
    pltpu.async_copy(adst_hbm.at[pl.ds(0, NT)], adst_v, sem_g[1])

    zeros16 = jnp.zeros((16,), _f32)
    lanes = lax.iota(_i32, 16)
    e16s = [j2 * 16 + lanes for j2 in range(GRP // 16)]

    def _zr(c, cc):
        c16 = jnp.full((16,), c, _i32)
        for j2 in range(GRP // 16):
            plsc.store_scatter(rows[0], [e16s[j2], c16], zeros16)
        return cc

    lax.fori_loop(0, D, _zr, 0)
    for j2 in range(GRP // 16):
        exb[0][pl.ds(j2 * 16, 16)] = zeros16

    base_row = sid * OUT_PT
    for off, ln in _ZCHUNKS:
        pltpu.async_copy(rows[0].at[pl.ds(0, ln)],
                         out_sh.at[pl.ds(base_row + off, ln)], sem_sc[0])

    @pl.when(sid == NS - 1)
    def _ztail():
        pltpu.sync_copy(rows[0].at[pl.ds(0, 16)],
                        out_sh.at[pl.ds(NS * OUT_PT, 16)])

    s_base = sid * (NT // NS)
    for soff, sln in _SCHUNKS:
        pltpu.async_copy(exb[0].at[pl.ds(0, sln)],
                         s_sh.at[pl.ds(s_base + soff, sln)], sem_sc[1])

    for off, ln in _ZCHUNKS:
        pltpu.make_async_copy(rows[0].at[pl.ds(0, ln)],
                              out_sh.at[pl.ds(base_row + off, ln)],
                              sem_sc[0]).wait()
    for soff, sln in _SCHUNKS:
        pltpu.make_async_copy(exb[0].at[pl.ds(0, sln)],
                              s_sh.at[pl.ds(s_base + soff, sln)],
                              sem_sc[1]).wait()
    pltpu.make_async_copy(asrc_hbm.at[pl.ds(0, NT)], asrc_v, sem_g[0]).wait()
    pltpu.make_async_copy(adst_hbm.at[pl.ds(0, NT)], adst_v, sem_g[1]).wait()

    plsc.subcore_barrier()

    pltpu.sync_copy(srcsg_hbm.at[base_sg], srcg[0])
    pltpu.sync_copy(dstsg_hbm.at[base_sg], dstg[0])
    pltpu.async_copy(h_hbm.at[srcg[0].at[pl.ds(0, GRP)]], rows[0], sem_g[0])
    pltpu.async_copy(h_hbm.at[srcg[0].at[pl.ds(GRP, GRP)]], rows[1], sem_g[1])
    pltpu.async_copy(h_hbm.at[srcg[0].at[pl.ds(2 * GRP, GRP)]], rows[2],
                     sem_g[2])


    def _wait_scatter(q):
        pltpu.make_async_copy(rows[q], out_sh.at[dstg[0].at[0]],
                              sem_sc[q]).wait()

    def _wait_sscatter(q):
        pltpu.make_async_copy(exb[q], s_sh.at[dstg[0].at[0]],
                              sem_ss[q]).wait()

    def _body(i, carry):
        for s3 in range(3):
            b3 = s3
            nb3 = (s3 + 1) % 3
            s_dyn = 3 * i + s3
            for k in range(SG):
                b = k % 4
                g_dyn = s_dyn * SG + k
                ebase = base_sg * (SG * GRP)
                if s3 == 0 and k < 4:
                    @pl.when(i > 0)
                    def _():
                        _wait_sscatter(b)
                else:
                    _wait_sscatter(b)
                for j2 in range(GRP // 16):
                    s16 = srcg[b3][pl.ds(k * GRP + j2 * 16, 16)]
                    d16 = dstg[b3][k, pl.ds(j2 * 16, 16)]
                    av = plsc.load_gather(asrc_v, [s16])
                    bv = plsc.load_gather(adst_v, [d16])
                    e = av + bv
                    e = jnp.where(e > 0.0, e, 0.2 * e)
                    ex = jnp.exp(e)
                    eid = ebase + g_dyn * GRP + j2 * 16 + lanes
                    ex = jnp.where(eid < EDGES, ex, 0.0)
                    exb[b][pl.ds(j2 * 16, 16)] = ex
                pltpu.async_copy(exb[b], s_sh.at[dstg[b3].at[k]], sem_ss[b],
                                 add=True)
                if k == 0:
                    def _issue_idx():
                        pltpu.async_copy(srcsg_hbm.at[base_sg + s_dyn + 1],
                                         srcg[nb3], sem_ix[nb3])
                        pltpu.async_copy(dstsg_hbm.at[base_sg + s_dyn + 1],
                                         dstg[nb3], sem_ix[nb3])
                    if s3 == 2:
                        pl.when(i < last_i)(_issue_idx)
                    else:
                        _issue_idx()
                pltpu.make_async_copy(h_hbm.at[srcg[0].at[pl.ds(0, GRP)]],
                                      rows[b], sem_g[b]).wait()
                @plsc.parallel_loop(0, GRP, step=1, unroll=2)
                def _sc(e):
                    m = plsc.load_gather(exb[b], [jnp.full((16,), e, _i32)])
                    for j in range(D // 16):
                        rows[b][e, pl.ds(j * 16, 16)] = (
                            rows[b][e, pl.ds(j * 16, 16)] * m)
                pltpu.async_copy(rows[b], out_sh.at[dstg[b3].at[k]],
                                 sem_sc[b], add=True)
                if s3 == 0 and k == 0:
                    @pl.when(i > 0)
                    def _():
                        _wait_scatter(3)
                else:
                    _wait_scatter((k + 3) % 4)
                def _issue_gather():
                    if k == SG - 3:
                        pltpu.make_async_copy(srcsg_hbm.at[base_sg],
                                              srcg[nb3], sem_ix[nb3]).wait()
                        pltpu.make_async_copy(dstsg_hbm.at[base_sg],
                                              dstg[nb3], sem_ix[nb3]).wait()
                        idx_ref = srcg[nb3].at[pl.ds(0, GRP)]
                    elif k > SG - 3:
                        idx_ref = srcg[nb3].at[pl.ds((k - (SG - 3)) * GRP,
                                                     GRP)]
                    else:
                        idx_ref = srcg[b3].at[pl.ds((k + 3) * GRP, GRP)]
                    pltpu.async_copy(h_hbm.at[idx_ref], rows[(k + 3) % 4],
                                     sem_g[(k + 3) % 4])
                if s3 == 2 and k >= SG - 3:
                    pl.when(i < last_i)(_issue_gather)
                else:
                    _issue_gather()
        return carry

    lax.fori_loop(0, nsg3, _body, 0)

    for q in range(4):
        _wait_sscatter(q)
    _wait_scatter(3)
    plsc.subcore_barrier()

    @pl.when(sid == 0)
    def _ws():
        pltpu.sync_copy(s_sh, sparts_hbm.at[cid].at[pl.ds(0, NT)])

    for off, ln in _ZCHUNKS:
        pltpu.async_copy(out_sh.at[pl.ds(base_row + off, ln)],
                         parts_hbm.at[cid].at[pl.ds(base_row + off, ln)],
                         sem_sc[0])
    for off, ln in _ZCHUNKS:
        pltpu.make_async_copy(out_sh.at[pl.ds(base_row + off, ln)],
                              parts_hbm.at[cid].at[pl.ds(base_row + off, ln)],
                              sem_sc[0]).wait()

    @pl.when(sid == NS - 1)
    def _wtail():
        pltpu.sync_copy(out_sh.at[pl.ds(NS * OUT_PT, 16)],
                        parts_hbm.at[cid].at[pl.ds(NS * OUT_PT, 16)])


_edge_kernel = pl.kernel(
    _edge_body,
    out_type=[
        jax.ShapeDtypeStruct((NC, NP), _f32),
        jax.ShapeDtypeStruct((NC, NP, D), _f32),
    ],
    mesh=plsc.VectorSubcoreMesh(core_axis_name="c", subcore_axis_name="s"),
    compiler_params=pltpu.CompilerParams(needs_layout_passes=False),
    scratch_types=[
        [pltpu.VMEM((SG * GRP,), _i32) for _ in range(3)],
        [pltpu.VMEM((SG, GRP), _i32) for _ in range(3)],
        pltpu.VMEM((NT,), _f32),
        pltpu.VMEM((NT,), _f32),
        [pltpu.VMEM((GRP, D), _f32) for _ in range(4)],
        [pltpu.VMEM((GRP,), _f32) for _ in range(4)],
        pltpu.VMEM_SHARED((N, D), _f32),
        pltpu.VMEM_SHARED((NT,), _f32),
        [pltpu.SemaphoreType.DMA for _ in range(4)],
        [pltpu.SemaphoreType.DMA for _ in range(4)],
        [pltpu.SemaphoreType.DMA for _ in range(3)],
        [pltpu.SemaphoreType.DMA for _ in range(4)],
    ],
)


def kernel(x, adj, W1, att_src1, att_dst1, bias1, bn_gamma, bn_beta, bn_mean,
           bn_var, W2, att_src2, att_dst2, bias2):
    loop = jnp.arange(N, dtype=_i32)
    src = jnp.concatenate([adj[0].astype(_i32), loop])
    dst = jnp.concatenate([adj[1].astype(_i32), loop])
    src = jnp.pad(src, (0, EP - EDGES))
    dst = jnp.pad(dst, (0, EP - EDGES))
    srcsg = src.reshape(NS * TSG, SG * GRP)
    dstsg = dst.reshape(NS * TSG, SG, GRP)

    xp = jnp.pad(x, ((0, NP - N), (0, 0)))
    asv1 = att_src1.reshape(1, D)
    adv1 = att_dst1.reshape(1, D)
    asv2 = att_src2.reshape(1, D)
    adv2 = att_dst2.reshape(1, D)
    bn_scale = (bn_gamma / jnp.sqrt(bn_var + 1e-5)).reshape(1, D)
    bn_shift = bn_beta.reshape(1, D) - bn_mean.reshape(1, D) * bn_scale

    h1, as1, ad1 = _dense(xp, W1, asv1, adv1)
    sparts1, parts1 = _edge_kernel(h1, as1, ad1, srcsg, dstsg)
    h2, as2, ad2 = _combine_dense(parts1, sparts1, bias1.reshape(1, D),
                                  bn_scale, bn_shift, W2, asv2, adv2)
    sparts2, parts2 = _edge_kernel(h2, as2, ad2, srcsg, dstsg)
    out = _final(parts2, sparts2, bias2.reshape(1, D))
    return out[:N]
